# baseline (speedup 1.0000x reference)
import functools
import math

import jax
import jax.numpy as jnp
from jax.experimental import pallas as pl


_QSCALE = math.log2(math.e) / 8.0


def _prep_kernel(x_ref, wqk_ref, wv_ref, eye_ref, xb_ref, wt_ref):
    j = pl.program_id(0)
    eye = eye_ref[...]

    def xp(w):
        wt_ref[...] = jax.lax.dot_general(
            w, eye, (((0,), (0,)), ((), ())), preferred_element_type=jnp.float32
        ).astype(jnp.bfloat16)

    @pl.when(j == 0)
    def _():
        xp((wqk_ref[...] * _QSCALE).astype(jnp.bfloat16))

    @pl.when(j == 1)
    def _():
        xp(wqk_ref[...].astype(jnp.bfloat16))

    @pl.when(j == 2)
    def _():
        xp(wv_ref[...].astype(jnp.bfloat16))
        xb_ref[...] = x_ref[...].astype(jnp.bfloat16)


def _qkv_kernel(xb_ref, wt_ref, b_ref, out_ref):
    acc = jnp.dot(xb_ref[...], wt_ref[...], preferred_element_type=jnp.float32)
    out_ref[...] = (acc + b_ref[0]).astype(jnp.bfloat16)


def _attn_kernel(q_ref, k_ref, v_ref, wo_ref, bo_ref, out_ref):
    hp = pl.program_id(1)
    Dh = 64
    wo = wo_ref[...].astype(jnp.bfloat16)
    contrib = None
    for i in range(2):
        q = q_ref[:, i * Dh:(i + 1) * Dh]
        k = k_ref[:, i * Dh:(i + 1) * Dh]
        v = v_ref[:, i * Dh:(i + 1) * Dh]
        dots = jax.lax.dot_general(
            q, k, (((1,), (1,)), ((), ())), preferred_element_type=jnp.float32
        )
        m = jnp.max(dots, axis=-1, keepdims=True)
        e = jnp.exp2(dots - m).astype(jnp.bfloat16)
        v_ext = jnp.concatenate(
            [v, jnp.ones((v.shape[0], 64), jnp.bfloat16)], axis=1
        )
        o_ext = jnp.dot(e, v_ext, preferred_element_type=jnp.float32)
        o = o_ext[:, :Dh] / o_ext[:, Dh:Dh + 1]
        c = jax.lax.dot_general(
            o.astype(jnp.bfloat16), wo[:, i * Dh:(i + 1) * Dh],
            (((1,), (1,)), ((), ())), preferred_element_type=jnp.float32,
        )
        contrib = c if contrib is None else contrib + c

    @pl.when(hp == 0)
    def _():
        out_ref[...] = contrib + bo_ref[...]

    @pl.when(hp != 0)
    def _():
        out_ref[...] += contrib


@functools.partial(jax.jit, static_argnames=())
def kernel(x, Wqk, bqk, Wv, bv, Wo, bo, rot):
    del rot
    B, L, D = x.shape
    H = 16
    Dh = D // H
    x2 = x.reshape(L, D)

    ball = jnp.concatenate([bqk.at[:D].multiply(_QSCALE), bv]).reshape(3, 1, D)
    eye = jnp.eye(D, dtype=jnp.bfloat16)
    xb, Wt = pl.pallas_call(
        _prep_kernel,
        grid=(3,),
        in_specs=[
            pl.BlockSpec((L, D), lambda j: (0, 0)),
            pl.BlockSpec((D, D), lambda j: (jnp.minimum(j, 1), 0)),
            pl.BlockSpec((D, D), lambda j: (0, 0)),
            pl.BlockSpec((D, D), lambda j: (0, 0)),
        ],
        out_specs=[
            pl.BlockSpec((L, D), lambda j: (0, 0)),
            pl.BlockSpec((D, D), lambda j: (0, j)),
        ],
        out_shape=[
            jax.ShapeDtypeStruct((L, D), jnp.bfloat16),
            jax.ShapeDtypeStruct((D, 3 * D), jnp.bfloat16),
        ],
    )(x2, Wqk, Wv, eye)
    qkv = pl.pallas_call(
        _qkv_kernel,
        grid=(3,),
        in_specs=[
            pl.BlockSpec((L, D), lambda j: (0, 0)),
            pl.BlockSpec((D, D), lambda j: (0, j)),
            pl.BlockSpec((1, 1, D), lambda j: (j, 0, 0)),
        ],
        out_specs=pl.BlockSpec((L, D), lambda j: (0, j)),
        out_shape=jax.ShapeDtypeStruct((L, 3 * D), jnp.bfloat16),
    )(xb, Wt, ball)

    HP = H // 2
    bo2 = bo.reshape(1, D)
    QB = L
    out = pl.pallas_call(
        _attn_kernel,
        grid=(L // QB, HP),
        in_specs=[
            pl.BlockSpec((QB, 2 * Dh), lambda qb, hp: (qb, hp)),
            pl.BlockSpec((L, 2 * Dh), lambda qb, hp: (0, HP + hp)),
            pl.BlockSpec((L, 2 * Dh), lambda qb, hp: (0, 2 * HP + hp)),
            pl.BlockSpec((D, 2 * Dh), lambda qb, hp: (0, hp)),
            pl.BlockSpec((1, D), lambda qb, hp: (0, 0)),
        ],
        out_specs=pl.BlockSpec((QB, D), lambda qb, hp: (qb, 0)),
        out_shape=jax.ShapeDtypeStruct((L, D), jnp.float32),
    )(qkv, qkv, qkv, Wo, bo2)

    return out.reshape(B, L, D)


# speedup vs baseline: 1.0572x; 1.0572x over previous
import functools
import math

import jax
import jax.numpy as jnp
from jax.experimental import pallas as pl
from jax.experimental.pallas import tpu as pltpu


_QSCALE = math.log2(math.e) / 8.0


def _fused_kernel(x_ref, wq_ref, wk_ref, wv_ref, bq_ref, bk_ref, bv_ref,
                  wo_ref, bo_ref, out_ref, xb_ref):
    hp = pl.program_id(0)
    Dh = 64

    @pl.when(hp == 0)
    def _():
        xb_ref[...] = x_ref[...].astype(jnp.bfloat16)

    xb = xb_ref[...]
    wo = wo_ref[...].astype(jnp.bfloat16)

    def proj(w_ref, b_ref, scale=None):
        w = w_ref[...]
        if scale is not None:
            w = w * scale
        acc = jax.lax.dot_general(
            xb, w.astype(jnp.bfloat16), (((1,), (1,)), ((), ())),
            preferred_element_type=jnp.float32,
        )
        return (acc + b_ref[0]).astype(jnp.bfloat16)

    qp = proj(wq_ref, bq_ref, _QSCALE)
    kp = proj(wk_ref, bk_ref)
    vp = proj(wv_ref, bv_ref)

    contrib = None
    for i in range(2):
        q = qp[:, i * Dh:(i + 1) * Dh]
        k = kp[:, i * Dh:(i + 1) * Dh]
        v = vp[:, i * Dh:(i + 1) * Dh]
        dots = jax.lax.dot_general(
            q, k, (((1,), (1,)), ((), ())), preferred_element_type=jnp.float32
        )
        m = jnp.max(dots, axis=-1, keepdims=True)
        e = jnp.exp2(dots - m).astype(jnp.bfloat16)
        v_ext = jnp.concatenate(
            [v, jnp.ones((v.shape[0], 64), jnp.bfloat16)], axis=1
        )
        o_ext = jnp.dot(e, v_ext, preferred_element_type=jnp.float32)
        o = o_ext[:, :Dh] / o_ext[:, Dh:Dh + 1]
        c = jax.lax.dot_general(
            o.astype(jnp.bfloat16), wo[:, i * Dh:(i + 1) * Dh],
            (((1,), (1,)), ((), ())), preferred_element_type=jnp.float32,
        )
        contrib = c if contrib is None else contrib + c

    @pl.when(hp == 0)
    def _():
        out_ref[...] = contrib + bo_ref[...]

    @pl.when(hp != 0)
    def _():
        out_ref[...] += contrib


@functools.partial(jax.jit, static_argnames=())
def kernel(x, Wqk, bqk, Wv, bv, Wo, bo, rot):
    del rot
    B, L, D = x.shape
    H = 16
    Dh = D // H
    HP = H // 2
    x2 = x.reshape(L, D)

    ball = jnp.concatenate(
        [bqk.at[:D].multiply(_QSCALE), bv]
    ).reshape(3 * HP, 1, 2 * Dh)
    bo2 = bo.reshape(1, D)

    out = pl.pallas_call(
        _fused_kernel,
        grid=(HP,),
        in_specs=[
            pl.BlockSpec((L, D), lambda hp: (0, 0)),
            pl.BlockSpec((2 * Dh, D), lambda hp: (hp, 0)),
            pl.BlockSpec((2 * Dh, D), lambda hp: (HP + hp, 0)),
            pl.BlockSpec((2 * Dh, D), lambda hp: (hp, 0)),
            pl.BlockSpec((1, 1, 2 * Dh), lambda hp: (hp, 0, 0)),
            pl.BlockSpec((1, 1, 2 * Dh), lambda hp: (HP + hp, 0, 0)),
            pl.BlockSpec((1, 1, 2 * Dh), lambda hp: (2 * HP + hp, 0, 0)),
            pl.BlockSpec((D, 2 * Dh), lambda hp: (0, hp)),
            pl.BlockSpec((1, D), lambda hp: (0, 0)),
        ],
        out_specs=pl.BlockSpec((L, D), lambda hp: (0, 0)),
        out_shape=jax.ShapeDtypeStruct((L, D), jnp.float32),
        scratch_shapes=[pltpu.VMEM((L, D), jnp.bfloat16)],
    )(x2, Wqk, Wqk, Wv, ball, ball, ball, Wo, bo2)

    return out.reshape(B, L, D)


# speedup vs baseline: 1.1045x; 1.0448x over previous
import functools
import math

import jax
import jax.numpy as jnp
from jax.experimental import pallas as pl
from jax.experimental.pallas import tpu as pltpu


_QSCALE = math.log2(math.e) / 8.0
_L = 2048
_D = 1024
_HP = 8
_QB = _L // 2
_NPROJ = 6
_PCOLS = 3 * _D // _NPROJ


def _fused_kernel(x_ref, wqk_ref, wv_ref, b_ref, wo_ref, bo_ref,
                  out_ref, qkv_ref):
    j = pl.program_id(0)
    Dh = 64

    @pl.when(j < _NPROJ)
    def _():
        xb = x_ref[...].astype(jnp.bfloat16)
        w = jnp.where(j < 4, wqk_ref[...], wv_ref[...])
        w = jnp.where(j < 2, w * _QSCALE, w)
        acc = jax.lax.dot_general(
            xb, w.astype(jnp.bfloat16), (((1,), (1,)), ((), ())),
            preferred_element_type=jnp.float32,
        )
        qkv_ref[:, pl.ds(j * _PCOLS, _PCOLS)] = (
            acc + b_ref[0]
        ).astype(jnp.bfloat16)

    @pl.when(j >= _NPROJ)
    def _():
        t = j - _NPROJ
        hp = t % _HP
        qb = t // _HP
        wo = wo_ref[...].astype(jnp.bfloat16)
        qp = qkv_ref[pl.ds(qb * _QB, _QB), pl.ds(hp * 128, 128)]
        kp = qkv_ref[:, pl.ds(_D + hp * 128, 128)]
        vp = qkv_ref[:, pl.ds(2 * _D + hp * 128, 128)]
        contrib = None
        for i in range(2):
            q = qp[:, i * Dh:(i + 1) * Dh]
            k = kp[:, i * Dh:(i + 1) * Dh]
            v = vp[:, i * Dh:(i + 1) * Dh]
            dots = jax.lax.dot_general(
                q, k, (((1,), (1,)), ((), ())),
                preferred_element_type=jnp.float32,
            )
            m = jnp.max(dots, axis=-1, keepdims=True)
            e = jnp.exp2(dots - m).astype(jnp.bfloat16)
            v_ext = jnp.concatenate(
                [v, jnp.ones((v.shape[0], 64), jnp.bfloat16)], axis=1
            )
            o_ext = jnp.dot(e, v_ext, preferred_element_type=jnp.float32)
            o = o_ext[:, :Dh] / o_ext[:, Dh:Dh + 1]
            c = jax.lax.dot_general(
                o.astype(jnp.bfloat16), wo[:, i * Dh:(i + 1) * Dh],
                (((1,), (1,)), ((), ())), preferred_element_type=jnp.float32,
            )
            contrib = c if contrib is None else contrib + c

        @pl.when(hp == 0)
        def _():
            out_ref[...] = contrib + bo_ref[...]

        @pl.when(hp != 0)
        def _():
            out_ref[...] += contrib


@functools.partial(jax.jit, static_argnames=())
def kernel(x, Wqk, bqk, Wv, bv, Wo, bo, rot):
    del rot
    B, L, D = x.shape
    x2 = x.reshape(L, D)

    ball = jnp.concatenate(
        [bqk.at[:D].multiply(_QSCALE), bv]
    ).reshape(_NPROJ, 1, _PCOLS)
    bo2 = bo.reshape(1, D)

    out = pl.pallas_call(
        _fused_kernel,
        grid=(_NPROJ + 2 * _HP,),
        in_specs=[
            pl.BlockSpec((L, D), lambda j: (0, 0)),
            pl.BlockSpec((_PCOLS, D), lambda j: (jnp.clip(j, 0, 3), 0)),
            pl.BlockSpec((_PCOLS, D), lambda j: (jnp.clip(j - 4, 0, 1), 0)),
            pl.BlockSpec((1, 1, _PCOLS),
                         lambda j: (jnp.minimum(j, _NPROJ - 1), 0, 0)),
            pl.BlockSpec((D, 128),
                         lambda j: (0, jnp.maximum(j - _NPROJ, 0) % _HP)),
            pl.BlockSpec((1, D), lambda j: (0, 0)),
        ],
        out_specs=pl.BlockSpec(
            (_QB, D), lambda j: (jnp.maximum(j - _NPROJ, 0) // _HP, 0)
        ),
        out_shape=jax.ShapeDtypeStruct((L, D), jnp.float32),
        scratch_shapes=[pltpu.VMEM((_L, 3 * _D), jnp.bfloat16)],
    )(x2, Wqk, Wv, ball, Wo, bo2)

    return out.reshape(B, L, D)
